# baseline (speedup 1.0000x reference)
import functools

import jax
import jax.numpy as jnp
from jax import lax
from jax.experimental import pallas as pl
from jax.experimental.pallas import tpu as pltpu
from jax.experimental.pallas import tpu_sc as plsc

_VOCAB = 1000000
_DIM = 64
_B = 16384
_L = 50

_NUM_CORES = 2
_NUM_SUBCORES = 16
_NW = _NUM_CORES * _NUM_SUBCORES
_BPW = _B // _NW
_HALF = 2
_CB = _BPW // _HALF
_NBLK = _L * _HALF
_NBUF = 6


def _make_gather_kernel():
  mesh = plsc.VectorSubcoreMesh(core_axis_name="c", subcore_axis_name="s")

  @functools.partial(
      pl.kernel,
      mesh=mesh,
      out_type=jax.ShapeDtypeStruct((_L, _B, _DIM), jnp.float32),
      scratch_types=(
          [pltpu.VMEM((_L, _BPW), jnp.int32)]
          + [pltpu.VMEM((_CB, _DIM), jnp.float32)] * _NBUF
          + [pltpu.SemaphoreType.DMA] * (2 * _NBUF)
      ),
      compiler_params=pltpu.CompilerParams(use_tc_tiling_on_sc=False),
  )
  def gather_kernel(idx_hbm, table_hbm, out_hbm, idx_v, *bufs):
    wid = lax.axis_index("s") * _NUM_CORES + lax.axis_index("c")
    base_b = wid * _BPW
    rows = bufs[:_NBUF]
    sem_g = bufs[_NBUF:2 * _NBUF]
    sem_o = bufs[2 * _NBUF:]

    def refs(i, b):
      l, h = i // _HALF, i % _HALF
      idx_ref = idx_v.at[l, pl.ds(h * _CB, _CB)]
      out_ref = out_hbm.at[l, pl.ds(base_b + h * _CB, _CB)]
      return idx_ref, out_ref

    def start_gather(i, b):
      idx_ref, _ = refs(i, b)
      pltpu.async_copy(table_hbm.at[idx_ref], rows[b], sem_g[b])

    def wait_gather(i, b):
      idx_ref, _ = refs(i, b)
      pltpu.make_async_copy(table_hbm.at[idx_ref], rows[b], sem_g[b]).wait()

    def start_out(i, b):
      _, out_ref = refs(i, b)
      pltpu.async_copy(rows[b], out_ref, sem_o[b])

    def wait_out(i, b):
      _, out_ref = refs(i, b)
      pltpu.make_async_copy(rows[b], out_ref, sem_o[b]).wait()

    pltpu.sync_copy(idx_hbm.at[pl.ds(0, _L), pl.ds(base_b, _BPW)], idx_v)

    pref = _NBUF - 1
    for i in range(pref):
      start_gather(i, i)
    for i in range(_NBLK):
      b = i % _NBUF
      if i + pref < _NBLK:
        if i >= 1:
          wait_out(i - 1, (i + pref) % _NBUF)
        start_gather(i + pref, (i + pref) % _NBUF)
      wait_gather(i, b)
      start_out(i, b)
    for i in range(_NBLK - _NBUF, _NBLK):
      wait_out(i, i % _NBUF)

  return gather_kernel


_gather = _make_gather_kernel()


@jax.jit
def kernel(input, table):
  idx_t = input.T.astype(jnp.int32)
  k = _gather(idx_t, table)
  return k.transpose(1, 0, 2)
